# baseline (speedup 1.0000x reference)
import functools

import jax
import jax.numpy as jnp
from jax import lax
from jax.experimental import pallas as pl
from jax.experimental.pallas import tpu as pltpu
from jax.experimental.pallas import tpu_sc as plsc

NUM_FEATURES = 26
CAT_SIZE = 1000
D_EMBED = 128
BATCH = 16384
PAD_F = 32

NC = 2
NS = 16
NW = NC * NS
SAMP_W = BATCH // NW
SAMP_C = 8
N_GATH = SAMP_C * PAD_F // 128
N_PAIR = SAMP_W // (2 * SAMP_C)


def _sc_embedding_gather(x_cat, weight):
    mesh = plsc.VectorSubcoreMesh(core_axis_name="c", subcore_axis_name="s")

    @functools.partial(
        pl.kernel,
        mesh=mesh,
        out_type=jax.ShapeDtypeStruct((BATCH, NUM_FEATURES, D_EMBED), jnp.float32),
        scratch_types=[
            pltpu.VMEM((SAMP_C, NUM_FEATURES), jnp.int32),
            pltpu.VMEM((N_GATH, 128), jnp.int32),
            pltpu.VMEM((SAMP_C * PAD_F, D_EMBED), jnp.float32),
            pltpu.VMEM((SAMP_C * PAD_F, D_EMBED), jnp.float32),
            pltpu.SemaphoreType.DMA,
            pltpu.SemaphoreType.DMA,
            pltpu.SemaphoreType.DMA,
        ],
    )
    def body(x_hbm, w_hbm, out_hbm, idx_in, idx4, bufa, bufb, semg, semwa, semwb):
        wid = lax.axis_index("s") * NC + lax.axis_index("c")
        base = wid * SAMP_W
        off1 = lax.iota(jnp.int32, 16) * CAT_SIZE
        off2 = (lax.iota(jnp.int32, 16) + 10) * CAT_SIZE
        zeros = jnp.zeros((16,), jnp.int32)
        bufs = (bufa, bufb)
        semws = (semwa, semwb)

        def run_chunk(sb, buf, semw, first):
            pltpu.sync_copy(x_hbm.at[pl.ds(sb, SAMP_C)], idx_in)
            for i in range(SAMP_C):
                j, k = divmod(i, 4)
                v1 = idx_in[i, pl.ds(0, 16)]
                v2 = idx_in[i, pl.ds(10, 16)]
                idx4[j, pl.ds(32 * k + 16, 16)] = zeros
                idx4[j, pl.ds(32 * k, 16)] = v1 + off1
                idx4[j, pl.ds(32 * k + 10, 16)] = v2 + off2
            def drain():
                for i in range(SAMP_C):
                    pltpu.make_async_copy(
                        buf.at[pl.ds(i * PAD_F, NUM_FEATURES)],
                        out_hbm.at[sb + i],
                        semw,
                    ).wait()
            if first is None:
                drain()
            else:
                pl.when(first)(drain)
            gathers = [
                pltpu.async_copy(
                    w_hbm.at[idx4.at[j]], buf.at[pl.ds(j * 128, 128)], semg
                )
                for j in range(N_GATH)
            ]
            for g in gathers:
                g.wait()
            for i in range(SAMP_C):
                pltpu.async_copy(
                    buf.at[pl.ds(i * PAD_F, NUM_FEATURES)],
                    out_hbm.at[sb + i],
                    semw,
                )

        def pair_body(c, carry):
            sb0 = pl.multiple_of(base + (2 * c) * SAMP_C, SAMP_C)
            sb1 = pl.multiple_of(base + (2 * c + 1) * SAMP_C, SAMP_C)
            run_chunk(sb0, bufa, semwa, c > 0)
            run_chunk(sb1, bufb, semwb, c > 0)
            return carry

        lax.fori_loop(0, N_PAIR, pair_body, 0)

        tail0 = pl.multiple_of(base + SAMP_W - 2 * SAMP_C, SAMP_C)
        tail1 = pl.multiple_of(base + SAMP_W - SAMP_C, SAMP_C)
        for buf, semw, sb in ((bufa, semwa, tail0), (bufb, semwb, tail1)):
            for i in range(SAMP_C):
                pltpu.make_async_copy(
                    buf.at[pl.ds(i * PAD_F, NUM_FEATURES)],
                    out_hbm.at[sb + i],
                    semw,
                ).wait()

    return body(x_cat, weight)


def kernel(x_cat, weight):
    return _sc_embedding_gather(x_cat, weight)


# speedup vs baseline: 12.1401x; 12.1401x over previous
import functools

import jax
import jax.numpy as jnp
from jax import lax
from jax.experimental import pallas as pl
from jax.experimental.pallas import tpu as pltpu
from jax.experimental.pallas import tpu_sc as plsc

NUM_FEATURES = 26
CAT_SIZE = 1000
D_EMBED = 128
BATCH = 16384
TOTAL = BATCH * NUM_FEATURES

NC = 2
NS = 16
NW = NC * NS
PER_W = TOTAL // NW
SAMP_W = BATCH // NW
G_IDX = 104
CH_ROWS = 416
SAMP_C = CH_ROWS // NUM_FEATURES
N_CH = PER_W // CH_ROWS
N_GRP = PER_W // 16
ADJ_OUTER = 8
ADJ_INNER = N_GRP // ADJ_OUTER


def _sc_embedding_gather(x1d, weight):
    mesh = plsc.VectorSubcoreMesh(core_axis_name="c", subcore_axis_name="s")

    @functools.partial(
        pl.kernel,
        mesh=mesh,
        out_type=jax.ShapeDtypeStruct((BATCH, NUM_FEATURES, D_EMBED), jnp.float32),
        scratch_types=[
            pltpu.VMEM((PER_W,), jnp.int32),
            pltpu.VMEM((CH_ROWS, D_EMBED), jnp.float32),
            pltpu.VMEM((CH_ROWS, D_EMBED), jnp.float32),
            pltpu.SemaphoreType.DMA,
            pltpu.SemaphoreType.DMA,
            pltpu.SemaphoreType.DMA,
            pltpu.SemaphoreType.DMA,
        ],
    )
    def body(x_hbm, w_hbm, out_hbm, idx_all, bufa, bufb, semga, semgb, semwa, semwb):
        wid = lax.axis_index("s") * NC + lax.axis_index("c")
        base = pl.multiple_of(wid * PER_W, PER_W)
        sbase = wid * SAMP_W

        pltpu.sync_copy(x_hbm.at[pl.ds(base, PER_W)], idx_all)

        lane = lax.iota(jnp.int32, 16)
        offs = [
            lax.rem(lane + (16 * k) % NUM_FEATURES, NUM_FEATURES) * CAT_SIZE
            for k in range(13)
        ]

        def adj_body(c, carry):
            g0 = c * ADJ_INNER
            for k in range(ADJ_INNER):
                sl = pl.ds(pl.multiple_of((g0 + k) * 16, 16), 16)
                idx_all[sl] = idx_all[sl] + offs[k % 13]
            return carry

        lax.fori_loop(0, ADJ_OUTER, adj_body, 0)

        def run_chunk(n, buf, semg, semw, first):
            f0 = pl.multiple_of(n * CH_ROWS, CH_ROWS)
            sb = sbase + n * SAMP_C
            wb = lambda: pltpu.async_copy(
                buf.reshape(SAMP_C, NUM_FEATURES, D_EMBED),
                out_hbm.at[pl.ds(sb, SAMP_C)],
                semw,
            )
            drain = lambda: pltpu.make_async_copy(
                buf.reshape(SAMP_C, NUM_FEATURES, D_EMBED),
                out_hbm.at[pl.ds(sb, SAMP_C)],
                semw,
            ).wait()
            if first is None:
                drain()
            else:
                pl.when(first)(drain)
            gathers = [
                pltpu.async_copy(
                    w_hbm.at[idx_all.at[pl.ds(f0 + q * G_IDX, G_IDX)]],
                    buf.at[pl.ds(q * G_IDX, G_IDX)],
                    semg,
                )
                for q in range(4)
            ]
            for g in gathers:
                g.wait()
            wb()

        def pair_body(c, carry):
            run_chunk(2 * c, bufa, semga, semwa, c > 0)
            run_chunk(2 * c + 1, bufb, semgb, semwb, c > 0)
            return carry

        lax.fori_loop(0, N_CH // 2, pair_body, 0)

        for buf, semw, n in ((bufa, semwa, N_CH - 2), (bufb, semwb, N_CH - 1)):
            sb = sbase + n * SAMP_C
            pltpu.make_async_copy(
                buf.reshape(SAMP_C, NUM_FEATURES, D_EMBED),
                out_hbm.at[pl.ds(sb, SAMP_C)],
                semw,
            ).wait()

    return body(x1d, weight)


def kernel(x_cat, weight):
    return _sc_embedding_gather(x_cat.reshape(TOTAL), weight)


# speedup vs baseline: 12.1666x; 1.0022x over previous
import functools

import jax
import jax.numpy as jnp
from jax import lax
from jax.experimental import pallas as pl
from jax.experimental.pallas import tpu as pltpu
from jax.experimental.pallas import tpu_sc as plsc

NUM_FEATURES = 26
CAT_SIZE = 1000
D_EMBED = 128
BATCH = 16384
TOTAL = BATCH * NUM_FEATURES

NC = 2
NS = 16
NW = NC * NS
PER_W = TOTAL // NW
SAMP_W = BATCH // NW
G_IDX = 104
CH_ROWS = 416
SAMP_C = CH_ROWS // NUM_FEATURES
N_CH = PER_W // CH_ROWS
N_GRP = PER_W // 16
ADJ_OUTER = 8
ADJ_INNER = N_GRP // ADJ_OUTER


def _sc_embedding_gather(x1d, weight):
    mesh = plsc.VectorSubcoreMesh(core_axis_name="c", subcore_axis_name="s")

    @functools.partial(
        pl.kernel,
        mesh=mesh,
        out_type=jax.ShapeDtypeStruct((BATCH, NUM_FEATURES, D_EMBED), jnp.float32),
        compiler_params=pltpu.CompilerParams(use_tc_tiling_on_sc=True),
        scratch_types=[
            pltpu.VMEM((PER_W,), jnp.int32),
            pltpu.VMEM((CH_ROWS, D_EMBED), jnp.float32),
            pltpu.VMEM((CH_ROWS, D_EMBED), jnp.float32),
            pltpu.SemaphoreType.DMA,
            pltpu.SemaphoreType.DMA,
            pltpu.SemaphoreType.DMA,
            pltpu.SemaphoreType.DMA,
        ],
    )
    def body(x_hbm, w_hbm, out_hbm, idx_all, bufa, bufb, semga, semgb, semwa, semwb):
        wid = lax.axis_index("s") * NC + lax.axis_index("c")
        base = pl.multiple_of(wid * PER_W, PER_W)
        sbase = wid * SAMP_W

        pltpu.sync_copy(x_hbm.at[pl.ds(base, PER_W)], idx_all)

        lane = lax.iota(jnp.int32, 16)
        offs = [
            lax.rem(lane + (16 * k) % NUM_FEATURES, NUM_FEATURES) * CAT_SIZE
            for k in range(13)
        ]

        def adj_body(c, carry):
            g0 = c * ADJ_INNER
            for k in range(ADJ_INNER):
                sl = pl.ds(pl.multiple_of((g0 + k) * 16, 16), 16)
                idx_all[sl] = idx_all[sl] + offs[k % 13]
            return carry

        lax.fori_loop(0, ADJ_OUTER, adj_body, 0)

        def run_chunk(n, buf, semg, semw, first):
            f0 = pl.multiple_of(n * CH_ROWS, CH_ROWS)
            sb = sbase + n * SAMP_C
            wb = lambda: pltpu.async_copy(
                buf.reshape(SAMP_C, NUM_FEATURES, D_EMBED),
                out_hbm.at[pl.ds(sb, SAMP_C)],
                semw,
            )
            drain = lambda: pltpu.make_async_copy(
                buf.reshape(SAMP_C, NUM_FEATURES, D_EMBED),
                out_hbm.at[pl.ds(sb, SAMP_C)],
                semw,
            ).wait()
            if first is None:
                drain()
            else:
                pl.when(first)(drain)
            gathers = [
                pltpu.async_copy(
                    w_hbm.at[idx_all.at[pl.ds(f0 + q * G_IDX, G_IDX)]],
                    buf.at[pl.ds(q * G_IDX, G_IDX)],
                    semg,
                )
                for q in range(4)
            ]
            for g in gathers:
                g.wait()
            wb()

        def pair_body(c, carry):
            run_chunk(2 * c, bufa, semga, semwa, c > 0)
            run_chunk(2 * c + 1, bufb, semgb, semwb, c > 0)
            return carry

        lax.fori_loop(0, N_CH // 2, pair_body, 0)

        for buf, semw, n in ((bufa, semwa, N_CH - 2), (bufb, semwb, N_CH - 1)):
            sb = sbase + n * SAMP_C
            pltpu.make_async_copy(
                buf.reshape(SAMP_C, NUM_FEATURES, D_EMBED),
                out_hbm.at[pl.ds(sb, SAMP_C)],
                semw,
            ).wait()

    return body(x1d, weight)


def kernel(x_cat, weight):
    return _sc_embedding_gather(x_cat.reshape(TOTAL), weight)
